# baseline (speedup 1.0000x reference)
import functools
import math

import jax
import jax.numpy as jnp
from jax import lax
from jax.experimental import pallas as pl
from jax.experimental.pallas import tpu as pltpu
from jax.experimental.pallas import tpu_sc as plsc

_NC = 2
_NS = 16
_NW = _NC * _NS

_DIM = 64
_SCALE = 8.0
_LANES = 16

_IDX_PAD = 128
_ROWS_PER_CHUNK = 8


def _make_gather(n_b: int, n_sl: int):
    assert n_b % (_NW * _ROWS_PER_CHUNK) == 0
    b_per_w = n_b // _NW
    n_chunks = b_per_w // _ROWS_PER_CHUNK
    chunk_rows = _ROWS_PER_CHUNK * n_sl
    n_total = n_b * n_sl

    mesh = plsc.VectorSubcoreMesh(
        core_axis_name="c", subcore_axis_name="s",
        num_cores=_NC, num_subcores=_NS,
    )

    @functools.partial(
        pl.kernel,
        out_type=jax.ShapeDtypeStruct((n_total, _DIM), jnp.float32),
        mesh=mesh,
        scratch_types=[
            pltpu.VMEM((b_per_w, _IDX_PAD), jnp.int32),
            pltpu.VMEM((chunk_rows, _DIM), jnp.float32),
            pltpu.VMEM((chunk_rows, _DIM), jnp.float32),
            pltpu.SemaphoreType.DMA,
            pltpu.SemaphoreType.DMA,
            pltpu.SemaphoreType.DMA,
            pltpu.SemaphoreType.DMA,
        ],
        compiler_params=pltpu.CompilerParams(use_tc_tiling_on_sc=False),
    )
    def emb_kernel(table_hbm, idx_hbm, out_hbm,
                   idx_v, rows0, rows1, g0, g1, s0, s1):
        wid = lax.axis_index("s") * _NC + lax.axis_index("c")
        base = wid * b_per_w * n_sl
        rows = (rows0, rows1)
        gsem = (g0, g1)
        ssem = (s0, s1)

        pltpu.sync_copy(idx_hbm.at[pl.ds(wid * b_per_w, b_per_w)], idx_v)

        def fire_gathers(t):
            buf = rows[t % 2]
            sem = gsem[t % 2]
            handles = []
            for j in range(_ROWS_PER_CHUNK):
                r = t * _ROWS_PER_CHUNK + j
                handles.append(pltpu.async_copy(
                    table_hbm.at[idx_v.at[r, pl.ds(0, n_sl)]],
                    buf.at[pl.ds(j * n_sl, n_sl)],
                    sem,
                ))
            return handles

        def scale_chunk(t):
            buf = rows[t % 2]

            def body(i, carry):
                r0 = i * _ROWS_PER_CHUNK
                for r in range(_ROWS_PER_CHUNK):
                    for c in range(_DIM // _LANES):
                        sl = (r0 + r, pl.ds(c * _LANES, _LANES))
                        buf[sl] = buf[sl] * _SCALE
                return carry

            lax.fori_loop(0, chunk_rows // _ROWS_PER_CHUNK, body, 0)

        def fire_store(t):
            buf = rows[t % 2]
            return pltpu.async_copy(
                buf, out_hbm.at[pl.ds(base + t * chunk_rows, chunk_rows)],
                ssem[t % 2],
            )

        pending_g = fire_gathers(0)
        pending_s = [None, None]
        for t in range(n_chunks):
            for h in pending_g:
                h.wait()
            if t + 1 < n_chunks:
                prev = pending_s[(t + 1) % 2]
                if prev is not None:
                    prev.wait()
                    pending_s[(t + 1) % 2] = None
                pending_g = fire_gathers(t + 1)
            scale_chunk(t)
            pending_s[t % 2] = fire_store(t)
        for h in pending_s:
            if h is not None:
                h.wait()

    return emb_kernel


def kernel(input_vec, table):
    b, s = input_vec.shape
    s_pad = ((s + 7) // 8) * 8
    idx = jnp.pad(input_vec.astype(jnp.int32), ((0, 0), (0, _IDX_PAD - s)))
    flat = _make_gather(b, s_pad)(table, idx)
    return flat.reshape(b, s_pad, _DIM)[:, :s, :]


# speedup vs baseline: 1.7146x; 1.7146x over previous
import functools
import math

import jax
import jax.numpy as jnp
from jax import lax
from jax.experimental import pallas as pl
from jax.experimental.pallas import tpu as pltpu
from jax.experimental.pallas import tpu_sc as plsc

_NC = 2
_NS = 16
_NW = _NC * _NS

_DIM = 64
_SCALE = 8.0
_LANES = 16

_STREAM_ROWS = 128
_STREAMS_PER_CHUNK = 5
_CHUNK = _STREAM_ROWS * _STREAMS_PER_CHUNK
_ROWS_UNROLL = 8


def _make_gather(n_total: int):
    assert n_total % (_NW * _CHUNK) == 0
    per_w = n_total // _NW
    n_chunks = per_w // _CHUNK

    mesh = plsc.VectorSubcoreMesh(
        core_axis_name="c", subcore_axis_name="s",
        num_cores=_NC, num_subcores=_NS,
    )

    @functools.partial(
        pl.kernel,
        out_type=jax.ShapeDtypeStruct((n_total, _DIM), jnp.float32),
        mesh=mesh,
        scratch_types=[
            pltpu.VMEM((per_w,), jnp.int32),
            pltpu.VMEM((_CHUNK, _DIM), jnp.float32),
            pltpu.VMEM((_CHUNK, _DIM), jnp.float32),
            pltpu.SemaphoreType.DMA,
            pltpu.SemaphoreType.DMA,
            pltpu.SemaphoreType.DMA,
            pltpu.SemaphoreType.DMA,
        ],
        compiler_params=pltpu.CompilerParams(use_tc_tiling_on_sc=False),
    )
    def emb_kernel(table_hbm, idx_hbm, out_hbm,
                   idx_v, rows0, rows1, g0, g1, s0, s1):
        wid = lax.axis_index("s") * _NC + lax.axis_index("c")
        base = wid * per_w
        rows = (rows0, rows1)
        gsem = (g0, g1)
        ssem = (s0, s1)

        pltpu.sync_copy(idx_hbm.at[pl.ds(base, per_w)], idx_v)

        def fire_gathers(t):
            buf = rows[t % 2]
            sem = gsem[t % 2]
            handles = []
            for j in range(_STREAMS_PER_CHUNK):
                s = t * _STREAMS_PER_CHUNK + j
                handles.append(pltpu.async_copy(
                    table_hbm.at[idx_v.at[pl.ds(s * _STREAM_ROWS,
                                                _STREAM_ROWS)]],
                    buf.at[pl.ds(j * _STREAM_ROWS, _STREAM_ROWS)],
                    sem,
                ))
            return handles

        def scale_chunk(t):
            buf = rows[t % 2]

            def body(i, carry):
                r0 = i * _ROWS_UNROLL
                for r in range(_ROWS_UNROLL):
                    for c in range(_DIM // _LANES):
                        sl = (r0 + r, pl.ds(c * _LANES, _LANES))
                        buf[sl] = buf[sl] * _SCALE
                return carry

            lax.fori_loop(0, _CHUNK // _ROWS_UNROLL, body, 0)

        def fire_store(t):
            buf = rows[t % 2]
            return pltpu.async_copy(
                buf, out_hbm.at[pl.ds(base + t * _CHUNK, _CHUNK)],
                ssem[t % 2],
            )

        pending_g = fire_gathers(0)
        pending_s = [None, None]
        for t in range(n_chunks):
            for h in pending_g:
                h.wait()
            if t + 1 < n_chunks:
                prev = pending_s[(t + 1) % 2]
                if prev is not None:
                    prev.wait()
                    pending_s[(t + 1) % 2] = None
                pending_g = fire_gathers(t + 1)
            scale_chunk(t)
            pending_s[t % 2] = fire_store(t)
        for h in pending_s:
            if h is not None:
                h.wait()

    return emb_kernel


def kernel(input_vec, table):
    b, s = input_vec.shape
    n_total = b * s
    idx = input_vec.reshape(n_total).astype(jnp.int32)
    out = _make_gather(n_total)(table, idx)
    return out.reshape(b, s, _DIM)
